# baseline (speedup 1.0000x reference)
import functools

import jax
import jax.numpy as jnp
from jax import lax
from jax.experimental import pallas as pl
from jax.experimental.pallas import tpu as pltpu
from jax.experimental.pallas import tpu_sc as plsc

_H = 32
_W = 32
_D = 256
_B = 64
_B_TC = 48
_B_SC = _B - _B_TC


def _tc_kernel(row_ref, col_ref, pose_ref, p_out_ref, m_out_ref, m_buf, sem):
    d = col_ref.shape[1]
    rep, hw, _ = m_buf.shape
    h = _H
    w = hw // h
    B = _B_TC

    cc = col_ref[1 : w + 1, :]
    rr = row_ref[1 : h + 1, :]
    left = jnp.broadcast_to(cc[None, :, :], (h, w, d)).reshape(hw, d)
    right = jnp.broadcast_to(rr[:, None, :], (h, w, d)).reshape(hw, d)
    m = jnp.concatenate([left, right], axis=1)
    m_buf[...] = jnp.broadcast_to(m[None], (rep, hw, 2 * d))

    pe = pose_ref[0, :]
    p2 = jnp.concatenate([pe, pe])
    p_out_ref[...] = jnp.broadcast_to(p2[None, :], (_B, 2 * d))

    copies = [
        pltpu.make_async_copy(m_buf, m_out_ref.at[pl.ds(i, 1)], sem.at[i % 2])
        for i in range(B)
    ]
    for c in copies:
        c.start()
    for c in copies:
        c.wait()


_mesh = plsc.VectorSubcoreMesh(core_axis_name="c", subcore_axis_name="s")


@functools.partial(
    pl.kernel,
    mesh=_mesh,
    out_type=jax.ShapeDtypeStruct((_B_SC, _H * _W, 2 * _D), jnp.float32),
    scratch_types=[
        pltpu.VMEM((_W, 2 * _D), jnp.float32),
        pltpu.SemaphoreType.DMA,
        pltpu.SemaphoreType.DMA,
    ],
)
def _sc_emb(row_hbm, col_hbm, m_out, patbuf, sem_in, sem_out):
    nc = 2
    wid = lax.axis_index("s") * nc + lax.axis_index("c")
    yp1 = wid + 1

    fills = []
    for xi in range(_W):
        fills.append(
            pltpu.make_async_copy(col_hbm.at[xi + 1], patbuf.at[xi, pl.ds(0, _D)], sem_in)
        )
        fills.append(
            pltpu.make_async_copy(row_hbm.at[yp1], patbuf.at[xi, pl.ds(_D, _D)], sem_in)
        )
    for f in fills:
        f.start()
    for f in fills:
        f.wait()

    outs = [
        pltpu.make_async_copy(patbuf, m_out.at[b, pl.ds(wid * _W, _W)], sem_out)
        for b in range(_B_SC)
    ]
    for o in outs:
        o.start()
    for o in outs:
        o.wait()


def kernel(x, row_embed, col_embed, pose_token_embed):
    B = x.shape[0]
    h, w = x.shape[-2], x.shape[-1]
    d = col_embed.shape[1]

    p_emb, m_tc = pl.pallas_call(
        _tc_kernel,
        in_specs=[
            pl.BlockSpec(memory_space=pltpu.VMEM),
            pl.BlockSpec(memory_space=pltpu.VMEM),
            pl.BlockSpec(memory_space=pltpu.VMEM),
        ],
        out_specs=[
            pl.BlockSpec(memory_space=pltpu.VMEM),
            pl.BlockSpec(memory_space=pl.ANY),
        ],
        out_shape=[
            jax.ShapeDtypeStruct((B, 2 * d), jnp.float32),
            jax.ShapeDtypeStruct((_B_TC, h * w, 2 * d), jnp.float32),
        ],
        scratch_shapes=[
            pltpu.VMEM((1, h * w, 2 * d), jnp.float32),
            pltpu.SemaphoreType.DMA((2,)),
        ],
    )(row_embed, col_embed, pose_token_embed)
    m_sc = _sc_emb(row_embed, col_embed)
    m_bhwc = jnp.concatenate([m_tc, m_sc], axis=0)
    m_emb = m_bhwc.reshape(B, h, w, 2 * d).transpose(0, 3, 1, 2)
    return (p_emb, m_emb)


# speedup vs baseline: 3.4580x; 3.4580x over previous
import jax
import jax.numpy as jnp
from jax.experimental import pallas as pl
from jax.experimental.pallas import tpu as pltpu

_REP = 2


def _emb_kernel(row_ref, col_ref, pose_ref, p_out_ref, m_out_ref, m_buf, sem):
    d = col_ref.shape[1]
    rep, hw, _ = m_buf.shape
    h = 32
    w = hw // h
    B = m_out_ref.shape[0]

    cc = col_ref[1 : w + 1, :]
    rr = row_ref[1 : h + 1, :]
    left = jnp.broadcast_to(cc[None, :, :], (h, w, d)).reshape(hw, d)
    right = jnp.broadcast_to(rr[:, None, :], (h, w, d)).reshape(hw, d)
    m = jnp.concatenate([left, right], axis=1)
    m_buf[...] = jnp.broadcast_to(m[None], (rep, hw, 2 * d))

    pe = pose_ref[0, :]
    p2 = jnp.concatenate([pe, pe])
    p_out_ref[...] = jnp.broadcast_to(p2[None, :], (B, 2 * d))

    copies = [
        pltpu.make_async_copy(m_buf, m_out_ref.at[pl.ds(i * rep, rep)], sem.at[i % 4])
        for i in range(B // rep)
    ]
    for c in copies:
        c.start()
    for c in copies:
        c.wait()


def kernel(x, row_embed, col_embed, pose_token_embed):
    B = x.shape[0]
    h, w = x.shape[-2], x.shape[-1]
    d = col_embed.shape[1]

    p_emb, m_bhwc = pl.pallas_call(
        _emb_kernel,
        in_specs=[
            pl.BlockSpec(memory_space=pltpu.VMEM),
            pl.BlockSpec(memory_space=pltpu.VMEM),
            pl.BlockSpec(memory_space=pltpu.VMEM),
        ],
        out_specs=[
            pl.BlockSpec(memory_space=pltpu.VMEM),
            pl.BlockSpec(memory_space=pl.ANY),
        ],
        out_shape=[
            jax.ShapeDtypeStruct((B, 2 * d), jnp.float32),
            jax.ShapeDtypeStruct((B, h * w, 2 * d), jnp.float32),
        ],
        scratch_shapes=[
            pltpu.VMEM((_REP, h * w, 2 * d), jnp.float32),
            pltpu.SemaphoreType.DMA((4,)),
        ],
    )(row_embed, col_embed, pose_token_embed)
    m_emb = m_bhwc.reshape(B, h, w, 2 * d).transpose(0, 3, 1, 2)
    return (p_emb, m_emb)


# speedup vs baseline: 3.4844x; 1.0076x over previous
import jax
import jax.numpy as jnp
from jax.experimental import pallas as pl
from jax.experimental.pallas import tpu as pltpu

_REP = 1


def _emb_kernel(row_ref, col_ref, pose_ref, p_out_ref, m_out_ref, m_buf, sem):
    d = col_ref.shape[1]
    rep, hw, _ = m_buf.shape
    h = 32
    w = hw // h
    B = m_out_ref.shape[0]

    cc = col_ref[1 : w + 1, :]
    rr = row_ref[1 : h + 1, :]
    left = jnp.broadcast_to(cc[None, :, :], (h, w, d)).reshape(hw, d)
    right = jnp.broadcast_to(rr[:, None, :], (h, w, d)).reshape(hw, d)
    m = jnp.concatenate([left, right], axis=1)
    m_buf[...] = jnp.broadcast_to(m[None], (rep, hw, 2 * d))

    pe = pose_ref[0, :]
    p2 = jnp.concatenate([pe, pe])
    p_out_ref[...] = jnp.broadcast_to(p2[None, :], (B, 2 * d))

    copies = [
        pltpu.make_async_copy(m_buf, m_out_ref.at[pl.ds(i * rep, rep)], sem.at[i % 8])
        for i in range(B // rep)
    ]
    for c in copies:
        c.start()
    for c in copies:
        c.wait()


def kernel(x, row_embed, col_embed, pose_token_embed):
    B = x.shape[0]
    h, w = x.shape[-2], x.shape[-1]
    d = col_embed.shape[1]

    p_emb, m_bhwc = pl.pallas_call(
        _emb_kernel,
        in_specs=[
            pl.BlockSpec(memory_space=pltpu.VMEM),
            pl.BlockSpec(memory_space=pltpu.VMEM),
            pl.BlockSpec(memory_space=pltpu.VMEM),
        ],
        out_specs=[
            pl.BlockSpec(memory_space=pltpu.VMEM),
            pl.BlockSpec(memory_space=pl.ANY),
        ],
        out_shape=[
            jax.ShapeDtypeStruct((B, 2 * d), jnp.float32),
            jax.ShapeDtypeStruct((B, h * w, 2 * d), jnp.float32),
        ],
        scratch_shapes=[
            pltpu.VMEM((_REP, h * w, 2 * d), jnp.float32),
            pltpu.SemaphoreType.DMA((8,)),
        ],
    )(row_embed, col_embed, pose_token_embed)
    m_emb = m_bhwc.reshape(B, h, w, 2 * d).transpose(0, 3, 1, 2)
    return (p_emb, m_emb)


# speedup vs baseline: 3.4918x; 1.0021x over previous
import jax
import jax.numpy as jnp
from jax.experimental import pallas as pl
from jax.experimental.pallas import tpu as pltpu

_REP = 1


def _emb_kernel(row_ref, col_ref, pose_ref, p_out_ref, m_out_ref, m_buf, sem):
    d = col_ref.shape[1]
    rep, hw, _ = m_buf.shape
    h = 32
    w = hw // h
    B = m_out_ref.shape[0]

    cc = col_ref[1 : w + 1, :]
    rr = row_ref[1 : h + 1, :]
    left = jnp.broadcast_to(cc[None, :, :], (h, w, d)).reshape(hw, d)
    right = jnp.broadcast_to(rr[:, None, :], (h, w, d)).reshape(hw, d)
    m = jnp.concatenate([left, right], axis=1)
    m_buf[...] = jnp.broadcast_to(m[None], (rep, hw, 2 * d))

    pe = pose_ref[0, :]
    p2 = jnp.concatenate([pe, pe])
    p_out_ref[...] = jnp.broadcast_to(p2[None, :], (B, 2 * d))

    copies = [
        pltpu.make_async_copy(m_buf, m_out_ref.at[pl.ds(i * rep, rep)], sem.at[i % 2])
        for i in range(B // rep)
    ]
    for c in copies:
        c.start()
    for c in copies:
        c.wait()


def kernel(x, row_embed, col_embed, pose_token_embed):
    B = x.shape[0]
    h, w = x.shape[-2], x.shape[-1]
    d = col_embed.shape[1]

    p_emb, m_bhwc = pl.pallas_call(
        _emb_kernel,
        in_specs=[
            pl.BlockSpec(memory_space=pltpu.VMEM),
            pl.BlockSpec(memory_space=pltpu.VMEM),
            pl.BlockSpec(memory_space=pltpu.VMEM),
        ],
        out_specs=[
            pl.BlockSpec(memory_space=pltpu.VMEM),
            pl.BlockSpec(memory_space=pl.ANY),
        ],
        out_shape=[
            jax.ShapeDtypeStruct((B, 2 * d), jnp.float32),
            jax.ShapeDtypeStruct((B, h * w, 2 * d), jnp.float32),
        ],
        scratch_shapes=[
            pltpu.VMEM((_REP, h * w, 2 * d), jnp.float32),
            pltpu.SemaphoreType.DMA((2,)),
        ],
    )(row_embed, col_embed, pose_token_embed)
    m_emb = m_bhwc.reshape(B, h, w, 2 * d).transpose(0, 3, 1, 2)
    return (p_emb, m_emb)


# speedup vs baseline: 3.5194x; 1.0079x over previous
import functools

import jax
import jax.numpy as jnp
from jax.experimental import pallas as pl
from jax.experimental.pallas import tpu as pltpu

_REP = 1


def _emb_kernel(h, row_ref, col_ref, pose_ref, p_out_ref, m_out_ref, m_buf, sem):
    d = col_ref.shape[1]
    rep, hw, _ = m_buf.shape
    w = hw // h
    B = m_out_ref.shape[0]

    cc = col_ref[1 : w + 1, :]
    rr = row_ref[1 : h + 1, :]
    left = jnp.broadcast_to(cc[None, :, :], (h, w, d)).reshape(hw, d)
    right = jnp.broadcast_to(rr[:, None, :], (h, w, d)).reshape(hw, d)
    m = jnp.concatenate([left, right], axis=1)
    m_buf[...] = jnp.broadcast_to(m[None], (rep, hw, 2 * d))

    pe = pose_ref[0, :]
    p2 = jnp.concatenate([pe, pe])
    p_out_ref[...] = jnp.broadcast_to(p2[None, :], (B, 2 * d))

    copies = [
        pltpu.make_async_copy(m_buf, m_out_ref.at[pl.ds(i * rep, rep)], sem.at[i % 2])
        for i in range(B // rep)
    ]
    for c in copies:
        c.start()
    for c in copies:
        c.wait()


def kernel(x, row_embed, col_embed, pose_token_embed):
    B = x.shape[0]
    h, w = x.shape[-2], x.shape[-1]
    d = col_embed.shape[1]

    p_emb, m_bhwc = pl.pallas_call(
        functools.partial(_emb_kernel, h),
        in_specs=[
            pl.BlockSpec(memory_space=pltpu.VMEM),
            pl.BlockSpec(memory_space=pltpu.VMEM),
            pl.BlockSpec(memory_space=pltpu.VMEM),
        ],
        out_specs=[
            pl.BlockSpec(memory_space=pltpu.VMEM),
            pl.BlockSpec(memory_space=pl.ANY),
        ],
        out_shape=[
            jax.ShapeDtypeStruct((B, 2 * d), jnp.float32),
            jax.ShapeDtypeStruct((B, h * w, 2 * d), jnp.float32),
        ],
        scratch_shapes=[
            pltpu.VMEM((_REP, h * w, 2 * d), jnp.float32),
            pltpu.SemaphoreType.DMA((2,)),
        ],
    )(row_embed, col_embed, pose_token_embed)
    m_emb = m_bhwc.reshape(B, h, w, 2 * d).transpose(0, 3, 1, 2)
    return (p_emb, m_emb)


# speedup vs baseline: 3.5478x; 1.0081x over previous
import functools

import jax
import jax.numpy as jnp
from jax.experimental import pallas as pl
from jax.experimental.pallas import tpu as pltpu

_REP = 1


def _emb_kernel(h, row_ref, col_ref, pose_ref, p_out_ref, m_out_ref, m_buf, p_buf, sem):
    d = col_ref.shape[1]
    rep, hw, _ = m_buf.shape
    w = hw // h
    B = m_out_ref.shape[0]

    cc = col_ref[1 : w + 1, :]
    rr = row_ref[1 : h + 1, :]
    left = jnp.broadcast_to(cc[None, :, :], (h, w, d)).reshape(hw, d)
    right = jnp.broadcast_to(rr[:, None, :], (h, w, d)).reshape(hw, d)
    m = jnp.concatenate([left, right], axis=1)
    m_buf[...] = jnp.broadcast_to(m[None], (rep, hw, 2 * d))

    pe = pose_ref[0, :]
    p2 = jnp.concatenate([pe, pe])
    p_buf[...] = jnp.broadcast_to(p2[None, :], (B, 2 * d))

    copies = [
        pltpu.make_async_copy(m_buf, m_out_ref.at[pl.ds(i * rep, rep)], sem.at[i % 2])
        for i in range(B // rep)
    ]
    copies.append(pltpu.make_async_copy(p_buf, p_out_ref, sem.at[0]))
    for c in copies:
        c.start()
    for c in copies:
        c.wait()


def kernel(x, row_embed, col_embed, pose_token_embed):
    B = x.shape[0]
    h, w = x.shape[-2], x.shape[-1]
    d = col_embed.shape[1]

    p_emb, m_bhwc = pl.pallas_call(
        functools.partial(_emb_kernel, h),
        in_specs=[
            pl.BlockSpec(memory_space=pltpu.VMEM),
            pl.BlockSpec(memory_space=pltpu.VMEM),
            pl.BlockSpec(memory_space=pltpu.VMEM),
        ],
        out_specs=[
            pl.BlockSpec(memory_space=pl.ANY),
            pl.BlockSpec(memory_space=pl.ANY),
        ],
        out_shape=[
            jax.ShapeDtypeStruct((B, 2 * d), jnp.float32),
            jax.ShapeDtypeStruct((B, h * w, 2 * d), jnp.float32),
        ],
        scratch_shapes=[
            pltpu.VMEM((_REP, h * w, 2 * d), jnp.float32),
            pltpu.VMEM((B, 2 * d), jnp.float32),
            pltpu.SemaphoreType.DMA((2,)),
        ],
    )(row_embed, col_embed, pose_token_embed)
    m_emb = m_bhwc.reshape(B, h, w, 2 * d).transpose(0, 3, 1, 2)
    return (p_emb, m_emb)
